# baseline (speedup 1.0000x reference)
import functools

import jax
import jax.numpy as jnp
from jax import lax
from jax.experimental import pallas as pl
from jax.experimental.pallas import tpu as pltpu
from jax.experimental.pallas import tpu_sc as plsc

V = 16384
HID = 64
OUTF = 64

NC = 2
NS = 16
NW = NC * NS
C = 128
SUP = 8
DG = 4
DS = 2
ROWS_PER_TILE = V // NS

_GD = lax.GatherDimensionNumbers(
    offset_dims=(), collapsed_slice_dims=(0,), start_index_map=(0,))


def _splat(vv, j):
    idx = jnp.full((16, 1), j, dtype=jnp.int32)
    return lax.gather(vv, idx, _GD, (1,),
                      mode=lax.GatherScatterMode.PROMISE_IN_BOUNDS)


def _sc_spmm(pk, vals, w_all, zeros_vh, nch):
    mesh = plsc.VectorSubcoreMesh(core_axis_name="c", subcore_axis_name="s")

    @functools.partial(
        pl.kernel,
        out_type=jax.ShapeDtypeStruct((NC, V, HID), jnp.float32),
        mesh=mesh,
        scratch_types=(
            [pltpu.VMEM((SUP, 2, C), jnp.int32) for _ in range(2)]
            + [pltpu.VMEM((SUP * C,), jnp.float32) for _ in range(2)]
            + [pltpu.VMEM((C, HID), jnp.float32) for _ in range(DG)]
            + [pltpu.VMEM((C, HID), jnp.float32) for _ in range(DS)]
            + [pltpu.VMEM_SHARED((V, HID), jnp.float32)]
            + [pltpu.SemaphoreType.DMA for _ in range(2 + DG + DS)]
        ),
        compiler_params=pltpu.CompilerParams(use_tc_tiling_on_sc=False),
    )
    def k(pk_hbm, vals_hbm, w_hbm, z_hbm, out_hbm,
          p0, p1, v0, v1, g0, g1, g2, g3, s0, s1, h_sh,
          ip0, ip1, ig0, ig1, ig2, ig3, is0, is1):
        cid = lax.axis_index("c")
        sid = lax.axis_index("s")
        wid = sid * NC + cid
        nsup = nch // SUP
        sbase = wid * nsup
        cbase = wid * nch

        pk_b = (p0, p1)
        vals_b = (v0, v1)
        gath_b = (g0, g1, g2, g3)
        scl_b = (s0, s1)
        isem = (ip0, ip1)
        gsem = (ig0, ig1, ig2, ig3)
        ssem = (is0, is1)

        pltpu.sync_copy(z_hbm.at[pl.ds(sid * ROWS_PER_TILE, ROWS_PER_TILE)],
                        h_sh.at[pl.ds(sid * ROWS_PER_TILE, ROWS_PER_TILE)])
        plsc.subcore_barrier()

        def issue_super(S, t):
            pltpu.async_copy(pk_hbm.at[sbase + S], pk_b[t], isem[t])
            pltpu.async_copy(vals_hbm.at[pl.ds((cbase + S * SUP) * C, SUP * C)],
                             vals_b[t], isem[t])

        def wait_super(t):
            pltpu.make_async_copy(pk_hbm.at[0], pk_b[t], isem[t]).wait()
            pltpu.make_async_copy(vals_hbm.at[pl.ds(0, SUP * C)], vals_b[t],
                                  isem[t]).wait()

        def issue_gather(sp, j, g):
            pltpu.async_copy(w_hbm.at[pk_b[sp].at[j, 1]], gath_b[g], gsem[g])

        def wait_gather(sp, j, g):
            pltpu.make_async_copy(w_hbm.at[pk_b[sp].at[j, 1]], gath_b[g],
                                  gsem[g]).wait()

        def issue_scatter(sp, j, b):
            pltpu.async_copy(scl_b[b], h_sh.at[pk_b[sp].at[j, 0]], ssem[b],
                             add=True)

        def wait_scatter(sp, j, b):
            pltpu.make_async_copy(scl_b[b], h_sh.at[pk_b[sp].at[j, 0]],
                                  ssem[b]).wait()

        def scale(sp, j, g, b):
            vb = vals_b[sp]
            gb = gath_b[g]
            sb = scl_b[b]

            def g_body(gi, carry):
                vv = vb[pl.ds(j * C + gi * 16, 16)]
                for jj in range(16):
                    spl = _splat(vv, jj)
                    e = gi * 16 + jj
                    for q in range(HID // 16):
                        sl = pl.ds(q * 16, 16)
                        sb[e, sl] = gb[e, sl] * spl
                return carry

            lax.fori_loop(0, C // 16, g_body, 0)

        def sub(S, j, sp, first=False, skip_si=False, skip_g=False):
            g, b = j % 4, j % 2
            wait_gather(sp, j, g)
            if not first:
                pj, psp = (j - 1, sp) if j > 0 else (SUP - 1, 1 - sp)
                wait_scatter(psp, pj, (j - 1) % 2)
            if j == 0 and not skip_si:
                issue_super(S + 1, 1 - sp)
            if not skip_g:
                if j < SUP - 3:
                    issue_gather(sp, j + 3, (j + 3) % 4)
                else:
                    if j == SUP - 3:
                        wait_super(1 - sp)
                    issue_gather(1 - sp, j - (SUP - 3), (j + 3) % 4)
            scale(sp, j, g, b)
            issue_scatter(sp, j, b)

        issue_super(0, 0)
        issue_super(1, 1)
        wait_super(0)
        for i in range(3):
            issue_gather(0, i, i)

        for j in range(SUP):
            sub(0, j, 0, first=(j == 0), skip_si=True)
        for j in range(SUP):
            sub(1, j, 1)

        def pair(m, carry):
            S = 2 + 2 * m
            for j in range(SUP):
                sub(S, j, 0)
            for j in range(SUP):
                sub(S + 1, j, 1)
            return carry

        lax.fori_loop(0, (nsup - 3) // 2, pair, 0)

        for j in range(SUP):
            sub(nsup - 1, j, 0, skip_si=True, skip_g=(j >= SUP - 3))
        wait_scatter(0, SUP - 1, (SUP - 1) % 2)
        plsc.subcore_barrier()

        pltpu.sync_copy(h_sh.at[pl.ds(sid * ROWS_PER_TILE, ROWS_PER_TILE)],
                        out_hbm.at[cid, pl.ds(sid * ROWS_PER_TILE, ROWS_PER_TILE)])

    return k(pk, vals, w_all, zeros_vh)


def _tc_project(x2, h0, h1, fc_w, fc_b2):
    M = x2.shape[0]
    KB = 2048
    nk = V // KB

    def body(x_ref, h0_ref, h1_ref, w_ref, b_ref, o_ref, acc_ref):
        kc = pl.program_id(0)

        @pl.when(kc == 0)
        def _():
            acc_ref[...] = jnp.zeros_like(acc_ref)

        h = h0_ref[...] + h1_ref[...]
        acc_ref[...] += jnp.dot(x_ref[...], h,
                                preferred_element_type=jnp.float32)

        @pl.when(kc == nk - 1)
        def _():
            o_ref[...] = (jnp.dot(acc_ref[...], w_ref[...],
                                  preferred_element_type=jnp.float32)
                          + b_ref[...])

    return pl.pallas_call(
        body,
        grid=(nk,),
        in_specs=[
            pl.BlockSpec((M, KB), lambda kc: (0, kc)),
            pl.BlockSpec((KB, HID), lambda kc: (kc, 0)),
            pl.BlockSpec((KB, HID), lambda kc: (kc, 0)),
            pl.BlockSpec((HID, OUTF), lambda kc: (0, 0)),
            pl.BlockSpec((1, OUTF), lambda kc: (0, 0)),
        ],
        out_specs=pl.BlockSpec((M, OUTF), lambda kc: (0, 0)),
        out_shape=jax.ShapeDtypeStruct((M, OUTF), jnp.float32),
        scratch_shapes=[pltpu.VMEM((M, OUTF), jnp.float32)],
        compiler_params=pltpu.CompilerParams(
            dimension_semantics=("arbitrary",)),
    )(x2, h0, h1, fc_w, fc_b2)


def kernel(adj0_indices, adj0_values, adj1_indices, adj1_values,
           adj2_indices, adj2_values, x_dv, W0_vh, W1_vh, W2_vh, fc_w, fc_b):
    rows = jnp.concatenate(
        [adj0_indices[0], adj1_indices[0], adj2_indices[0]])
    cols = jnp.concatenate(
        [adj0_indices[1], adj1_indices[1] + V, adj2_indices[1] + 2 * V])
    vals = jnp.concatenate([adj0_values, adj1_values, adj2_values])

    total = rows.shape[0]
    wave = NW * C
    nsup = -(-total // (wave * SUP))
    if nsup % 2 == 0:
        nsup += 1
    nch = nsup * SUP
    ne = nch * wave
    pad = ne - total
    rows = jnp.pad(rows, (0, pad))
    cols = jnp.pad(cols, (0, pad))
    vals = jnp.pad(vals, (0, pad))

    pk = jnp.stack([rows.reshape(-1, C), cols.reshape(-1, C)],
                   axis=1).reshape(-1, SUP, 2, C)


    w_all = jnp.concatenate([W0_vh, W1_vh, W2_vh], axis=0)
    zeros_vh = jnp.zeros((V, HID), jnp.float32)

    hp = _sc_spmm(pk, vals, w_all, zeros_vh, nch)

    b, d, _ = x_dv.shape
    x2 = x_dv.reshape(b * d, V)
    out = _tc_project(x2, hp[0], hp[1], fc_w, fc_b.reshape(1, OUTF))
    return out.reshape(b, d, OUTF)


# speedup vs baseline: 1.2834x; 1.2834x over previous
import functools

import jax
import jax.numpy as jnp
from jax import lax
from jax.experimental import pallas as pl
from jax.experimental.pallas import tpu as pltpu
from jax.experimental.pallas import tpu_sc as plsc

V = 16384
HID = 64
OUTF = 64

NC = 2
NS = 16
NW = NC * NS
C = 128
DI = 8
DG = 4
DS = 2
ROWS_PER_TILE = V // NS

_GD = lax.GatherDimensionNumbers(
    offset_dims=(), collapsed_slice_dims=(0,), start_index_map=(0,))


def _splat(vv, j):
    idx = jnp.full((16, 1), j, dtype=jnp.int32)
    return lax.gather(vv, idx, _GD, (1,),
                      mode=lax.GatherScatterMode.PROMISE_IN_BOUNDS)


def _sc_spmm(pk, vals, w_all, zeros_vh, nch):
    mesh = plsc.VectorSubcoreMesh(core_axis_name="c", subcore_axis_name="s")

    @functools.partial(
        pl.kernel,
        out_type=jax.ShapeDtypeStruct((NC, V, HID), jnp.float32),
        mesh=mesh,
        scratch_types=(
            [pltpu.VMEM((2, C), jnp.int32) for _ in range(DI)]
            + [pltpu.VMEM((C,), jnp.float32) for _ in range(DI)]
            + [pltpu.VMEM((C, HID), jnp.float32) for _ in range(DG)]
            + [pltpu.VMEM((C, HID), jnp.float32) for _ in range(DS)]
            + [pltpu.VMEM_SHARED((V, HID), jnp.float32)]
            + [pltpu.SemaphoreType.DMA for _ in range(DI + DG + DS)]
        ),
        compiler_params=pltpu.CompilerParams(use_tc_tiling_on_sc=False),
    )
    def k(pk_hbm, vals_hbm, w_hbm, z_hbm, out_hbm,
          p0, p1, p2, p3, p4, p5, p6, p7,
          v0, v1, v2, v3, v4, v5, v6, v7,
          g0, g1, g2, g3, s0, s1, h_sh,
          ip0, ip1, ip2, ip3, ip4, ip5, ip6, ip7,
          ig0, ig1, ig2, ig3, is0, is1):
        cid = lax.axis_index("c")
        sid = lax.axis_index("s")
        wid = sid * NC + cid
        cbase = wid * nch

        pk_b = (p0, p1, p2, p3, p4, p5, p6, p7)
        vals_b = (v0, v1, v2, v3, v4, v5, v6, v7)
        gath_b = (g0, g1, g2, g3)
        scl_b = (s0, s1)
        isem = (ip0, ip1, ip2, ip3, ip4, ip5, ip6, ip7)
        gsem = (ig0, ig1, ig2, ig3)
        ssem = (is0, is1)

        pltpu.sync_copy(z_hbm.at[pl.ds(sid * ROWS_PER_TILE, ROWS_PER_TILE)],
                        h_sh.at[pl.ds(sid * ROWS_PER_TILE, ROWS_PER_TILE)])
        plsc.subcore_barrier()

        def issue_idx(kc, t):
            pltpu.async_copy(pk_hbm.at[cbase + kc], pk_b[t], isem[t])
            pltpu.async_copy(vals_hbm.at[pl.ds((cbase + kc) * C, C)],
                             vals_b[t], isem[t])

        def wait_idx(t):
            pltpu.make_async_copy(pk_hbm.at[0], pk_b[t], isem[t]).wait()
            pltpu.make_async_copy(vals_hbm.at[pl.ds(0, C)], vals_b[t],
                                  isem[t]).wait()

        def issue_gather(t, g):
            pltpu.async_copy(w_hbm.at[pk_b[t].at[1]], gath_b[g], gsem[g])

        def wait_gather(t, g):
            pltpu.make_async_copy(w_hbm.at[pk_b[t].at[1]], gath_b[g],
                                  gsem[g]).wait()

        def issue_scatter(t, b):
            pltpu.async_copy(scl_b[b], h_sh.at[pk_b[t].at[0]], ssem[b],
                             add=True)

        def wait_scatter(t, b):
            pltpu.make_async_copy(scl_b[b], h_sh.at[pk_b[t].at[0]],
                                  ssem[b]).wait()

        def scale(t, g, b):
            vb = vals_b[t]
            gb = gath_b[g]
            sb = scl_b[b]

            def g_body(gi, carry):
                vv = vb[pl.ds(gi * 16, 16)]
                for j in range(16):
                    sp = _splat(vv, j)
                    e = gi * 16 + j
                    for q in range(HID // 16):
                        sl = pl.ds(q * 16, 16)
                        sb[e, sl] = gb[e, sl] * sp
                return carry

            lax.fori_loop(0, C // 16, g_body, 0)

        def half(kc, u, first=False, steady=False):
            t, g, b = u % DI, u % DG, u % DS
            wait_gather(t, g)
            if not first:
                wait_scatter((u - 1) % DI, (u - 1) % DS)
            if steady or kc + 5 < nch:
                issue_idx(kc + 5, (u + 5) % DI)
            if steady or kc + 3 < nch:
                wait_idx((u + 3) % DI)
                issue_gather((u + 3) % DI, (u + 3) % DG)
            scale(t, g, b)
            issue_scatter(t, b)

        for i in range(5):
            issue_idx(i, i)
        for i in range(3):
            wait_idx(i)
            issue_gather(i, i)
        half(0, 0, first=True)
        half(1, 1)

        def eight(kk, carry):
            k0 = 2 + kk * DI
            for u in range(DI):
                half(k0 + u, 2 + u, steady=True)
            return carry

        lax.fori_loop(0, (nch - 7) // DI, eight, 0)

        for kc in range(nch - 5, nch):
            half(kc, kc % DI)
        wait_scatter((nch - 1) % DI, (nch - 1) % DS)
        plsc.subcore_barrier()

        pltpu.sync_copy(h_sh.at[pl.ds(sid * ROWS_PER_TILE, ROWS_PER_TILE)],
                        out_hbm.at[cid, pl.ds(sid * ROWS_PER_TILE, ROWS_PER_TILE)])

    return k(pk, vals, w_all, zeros_vh)


def _tc_project(x2, h0, h1, fc_w, fc_b2):
    M = x2.shape[0]
    KB = 2048
    nk = V // KB

    def body(x_ref, h0_ref, h1_ref, w_ref, b_ref, o_ref, acc_ref):
        kc = pl.program_id(0)

        @pl.when(kc == 0)
        def _():
            acc_ref[...] = jnp.zeros_like(acc_ref)

        h = h0_ref[...] + h1_ref[...]
        acc_ref[...] += jnp.dot(x_ref[...], h,
                                preferred_element_type=jnp.float32)

        @pl.when(kc == nk - 1)
        def _():
            o_ref[...] = (jnp.dot(acc_ref[...], w_ref[...],
                                  preferred_element_type=jnp.float32)
                          + b_ref[...])

    return pl.pallas_call(
        body,
        grid=(nk,),
        in_specs=[
            pl.BlockSpec((M, KB), lambda kc: (0, kc)),
            pl.BlockSpec((KB, HID), lambda kc: (kc, 0)),
            pl.BlockSpec((KB, HID), lambda kc: (kc, 0)),
            pl.BlockSpec((HID, OUTF), lambda kc: (0, 0)),
            pl.BlockSpec((1, OUTF), lambda kc: (0, 0)),
        ],
        out_specs=pl.BlockSpec((M, OUTF), lambda kc: (0, 0)),
        out_shape=jax.ShapeDtypeStruct((M, OUTF), jnp.float32),
        scratch_shapes=[pltpu.VMEM((M, OUTF), jnp.float32)],
        compiler_params=pltpu.CompilerParams(
            dimension_semantics=("arbitrary",)),
    )(x2, h0, h1, fc_w, fc_b2)


def kernel(adj0_indices, adj0_values, adj1_indices, adj1_values,
           adj2_indices, adj2_values, x_dv, W0_vh, W1_vh, W2_vh, fc_w, fc_b):
    rows = jnp.concatenate(
        [adj0_indices[0], adj1_indices[0], adj2_indices[0]])
    cols = jnp.concatenate(
        [adj0_indices[1], adj1_indices[1] + V, adj2_indices[1] + 2 * V])
    vals = jnp.concatenate([adj0_values, adj1_values, adj2_values])

    total = rows.shape[0]
    wave = NW * C
    nch = -(-total // wave)
    nch += (7 - nch) % 8
    ne = nch * wave
    pad = ne - total
    rows = jnp.pad(rows, (0, pad))
    cols = jnp.pad(cols, (0, pad))
    vals = jnp.pad(vals, (0, pad))

    pk = jnp.stack([rows.reshape(-1, C), cols.reshape(-1, C)], axis=1)

    w_all = jnp.concatenate([W0_vh, W1_vh, W2_vh], axis=0)
    zeros_vh = jnp.zeros((V, HID), jnp.float32)

    hp = _sc_spmm(pk, vals, w_all, zeros_vh, nch)

    b, d, _ = x_dv.shape
    x2 = x_dv.reshape(b * d, V)
    out = _tc_project(x2, hp[0], hp[1], fc_w, fc_b.reshape(1, OUTF))
    return out.reshape(b, d, OUTF)
